# baseline (speedup 1.0000x reference)
import functools

import jax
import jax.numpy as jnp
from jax import lax
from jax.experimental import pallas as pl
from jax.experimental.pallas import tpu as pltpu
from jax.experimental.pallas import tpu_sc as plsc

VOCAB = 1000000
D = 64
JOINT = 100000

NC, NS, L = 2, 16, 16
NW = NC * NS
CH = 128
N_CH = 25
B_PER_W = CH * N_CH
B_PAD = B_PER_W * NW

_mesh = plsc.VectorSubcoreMesh(core_axis_name="c", subcore_axis_name="s")


@functools.partial(
    pl.kernel,
    out_type=jax.ShapeDtypeStruct((NW, L), jnp.float32),
    mesh=_mesh,
    compiler_params=pltpu.CompilerParams(use_tc_tiling_on_sc=False),
    scratch_types=[
        pltpu.VMEM((B_PER_W,), jnp.int32),
        pltpu.VMEM((B_PER_W,), jnp.int32),
        pltpu.VMEM((CH, D), jnp.float32),
        pltpu.VMEM((CH, D), jnp.float32),
        pltpu.VMEM((CH, D), jnp.float32),
        pltpu.VMEM((CH, D), jnp.float32),
        pltpu.VMEM((L,), jnp.float32),
        pltpu.SemaphoreType.DMA,
        pltpu.SemaphoreType.DMA,
        pltpu.SemaphoreType.DMA,
        pltpu.SemaphoreType.DMA,
    ],
)
def _sc_gather_mse(wa_hbm, wb_hbm, ia_hbm, ib_hbm, out_hbm,
                   ia_v, ib_v, a0, a1, b0, b1, acc_v, sa0, sa1, sb0, sb1):
    wid = lax.axis_index("s") * NC + lax.axis_index("c")
    base = wid * B_PER_W
    pltpu.sync_copy(ia_hbm.at[pl.ds(base, B_PER_W)], ia_v)
    pltpu.sync_copy(ib_hbm.at[pl.ds(base, B_PER_W)], ib_v)

    abufs, bbufs = (a0, a1), (b0, b1)
    sas, sbs = (sa0, sa1), (sb0, sb1)

    def start(ch, p):
        ca = pltpu.async_copy(wa_hbm.at[ia_v.at[pl.ds(ch * CH, CH)]],
                              abufs[p], sas[p])
        cb = pltpu.async_copy(wb_hbm.at[ib_v.at[pl.ds(ch * CH, CH)]],
                              bbufs[p], sbs[p])
        return ca, cb

    def compute(p, accs):
        ab, bb = abufs[p], bbufs[p]

        def row(r, accs):
            new = []
            for j in range(D // L):
                av = ab[r, pl.ds(j * L, L)]
                bv = bb[r, pl.ds(j * L, L)]
                d = av - bv
                new.append(accs[j] + d * d)
            return tuple(new)

        return lax.fori_loop(0, CH, row, accs)

    accs = tuple(jnp.zeros((L,), jnp.float32) for _ in range(D // L))
    pending = start(0, 0)
    for ch in range(N_CH):
        p = ch % 2
        nxt = start(ch + 1, 1 - p) if ch + 1 < N_CH else None
        pending[0].wait()
        pending[1].wait()
        accs = compute(p, accs)
        pending = nxt

    acc_v[...] = (accs[0] + accs[1]) + (accs[2] + accs[3])
    pltpu.sync_copy(acc_v, out_hbm.at[wid])


def kernel(W_i2t, W_nmt, maps):
    idx_a = maps[:, 0].astype(jnp.int32)
    idx_b = maps[:, 1].astype(jnp.int32)
    pad = B_PAD - JOINT
    zeros = jnp.zeros((pad,), jnp.int32)
    idx_a = jnp.concatenate([idx_a, zeros])
    idx_b = jnp.concatenate([idx_b, zeros])
    partials = _sc_gather_mse(W_i2t, W_nmt, idx_a, idx_b)
    corr = jnp.sum((W_nmt[0, :] - W_i2t[0, :]) ** 2)
    total = jnp.sum(partials) - pad * corr
    return total / (JOINT * D)
